# baseline (speedup 1.0000x reference)
import functools

import jax
import jax.numpy as jnp
from jax import lax
from jax.experimental import pallas as pl
from jax.experimental.pallas import tpu as pltpu
from jax.experimental.pallas import tpu_sc as plsc

BATCH = 16384
NUM_FIELDS = 26
EMBED_DIM = 64
FIELD_SIZE = 100000
TOTAL = BATCH * NUM_FIELDS
TABLE_ROWS = NUM_FIELDS * FIELD_SIZE

_info = plsc.get_sparse_core_info()
NC = _info.num_cores
NS = _info.num_subcores
LANES = _info.num_lanes
NW = NC * NS
BROWS_PER_W = BATCH // NW
ROWS_PER_W = TOTAL // NW

BK = 1024
TC_GRID = -(-TABLE_ROWS // BK)
P_ROWS = TC_GRID * (BK // 2)

G_IDX = 128
N_GROUPS = ROWS_PER_W // G_IDX
OUT_PER_G = G_IDX // 2


def _pair_pack(table_t):
    def body(in_ref, out_ref):
        t = jnp.swapaxes(in_ref[...], 0, 1)
        t3 = t.reshape(BK // 2, 2, EMBED_DIM)
        out_ref[...] = jnp.concatenate([t3[:, 0, :], t3[:, 1, :]], axis=1)

    return pl.pallas_call(
        body,
        grid=(TC_GRID,),
        in_specs=[
            pl.BlockSpec((EMBED_DIM, BK), lambda g: (0, g)),
        ],
        out_specs=pl.BlockSpec((BK // 2, 2 * EMBED_DIM), lambda g: (g, 0)),
        out_shape=jax.ShapeDtypeStruct((P_ROWS, 2 * EMBED_DIM), jnp.float32),
    )(table_t)


def _sc_gather(xt, pairs):
    mesh = plsc.VectorSubcoreMesh(core_axis_name="c", subcore_axis_name="s")

    @functools.partial(
        pl.kernel,
        mesh=mesh,
        compiler_params=pltpu.CompilerParams(needs_layout_passes=False),
        out_type=jax.ShapeDtypeStruct((TOTAL // 2, 2 * EMBED_DIM), jnp.float32),
        scratch_types=[
            pltpu.VMEM((BROWS_PER_W,), jnp.int32),
            pltpu.VMEM((N_GROUPS, G_IDX), jnp.int32),
            pltpu.VMEM((ROWS_PER_W,), jnp.int32),
            pltpu.VMEM((2, G_IDX, 2 * EMBED_DIM), jnp.float32),
            pltpu.VMEM((2, OUT_PER_G, 2 * EMBED_DIM), jnp.float32),
            pltpu.SemaphoreType.DMA,
        ],
    )
    def k(xt_hbm, p_hbm, out_hbm, xcol, idxbuf, offbuf, rowbuf, outbuf, sem):
        wid = lax.axis_index("s") * NC + lax.axis_index("c")
        lane = lax.iota(jnp.int32, LANES)
        b0 = wid * BROWS_PER_W

        for f in range(NUM_FIELDS):
            pltpu.sync_copy(xt_hbm.at[f, pl.ds(b0, BROWS_PER_W)], xcol)

            def col_body(j, carry, f=f):
                v = xcol[pl.ds(j * LANES, LANES)]
                idx = lax.max(lax.min(v, FIELD_SIZE - 1), 0) + f * FIELD_SIZE
                p = (j * LANES + lane) * NUM_FIELDS + f
                grp = lax.shift_right_logical(p, 7)
                col = lax.bitwise_and(p, 127)
                plsc.store_scatter(
                    idxbuf, [grp, col], lax.shift_right_logical(idx, 1)
                )
                plsc.store_scatter(
                    offbuf,
                    [p],
                    lax.shift_left(lax.bitwise_and(idx, 1), 6),
                )
                return carry

            lax.fori_loop(0, BROWS_PER_W // LANES, col_body, 0)

        out_base = wid * (ROWS_PER_W // 2)
        qlane = lax.shift_right_logical(lane, 1)
        c2lane = lax.shift_left(lax.bitwise_and(lane, 1), 6)

        def fire(g, buf):
            return pltpu.async_copy(
                p_hbm.at[idxbuf.at[g]], rowbuf.at[buf], sem
            )

        def select_and_store(g, buf, copy):
            copy.wait()

            def u_body(u, carry):
                qbase = u * LANES
                coloff = offbuf[pl.ds(g * G_IDX + qbase, LANES)]
                qv = qbase + lane
                mv = lax.shift_right_logical(qbase, 1) + qlane
                cv = coloff
                c2v = c2lane
                for c0 in range(EMBED_DIM):
                    vals = plsc.load_gather(rowbuf.at[buf], [qv, cv + c0])
                    plsc.store_scatter(outbuf.at[buf], [mv, c2v + c0], vals)
                return carry

            lax.fori_loop(0, G_IDX // LANES, u_body, 0)
            pltpu.sync_copy(
                outbuf.at[buf],
                out_hbm.at[pl.ds(out_base + g * OUT_PER_G, OUT_PER_G)],
            )

        def chunk_body(d, carry):
            g0 = 2 * d
            c0 = fire(g0, 0)
            c1 = fire(g0 + 1, 1)
            select_and_store(g0, 0, c0)
            select_and_store(g0 + 1, 1, c1)
            return carry

        lax.fori_loop(0, N_GROUPS // 2, chunk_body, 0)

    return k(xt, pairs)


def kernel(x, table):
    pairs = _pair_pack(table.T)
    out = _sc_gather(x.T, pairs)
    return out.reshape(BATCH, NUM_FIELDS, EMBED_DIM)


# speedup vs baseline: 2.6055x; 2.6055x over previous
import functools

import jax
import jax.numpy as jnp
from jax import lax
from jax.experimental import pallas as pl
from jax.experimental.pallas import tpu as pltpu
from jax.experimental.pallas import tpu_sc as plsc

BATCH = 16384
NUM_FIELDS = 26
EMBED_DIM = 64
FIELD_SIZE = 100000
TOTAL = BATCH * NUM_FIELDS

_info = plsc.get_sparse_core_info()
NC = _info.num_cores
NS = _info.num_subcores
LANES = _info.num_lanes
NW = NC * NS
BROWS_PER_W = BATCH // NW

CHUNK = 256
STEPS = BROWS_PER_W // CHUNK


def _sc_gather(xt, table):
    mesh = plsc.VectorSubcoreMesh(core_axis_name="c", subcore_axis_name="s")

    @functools.partial(
        pl.kernel,
        mesh=mesh,
        compiler_params=pltpu.CompilerParams(needs_layout_passes=False),
        out_type=jax.ShapeDtypeStruct(
            (BATCH, NUM_FIELDS, EMBED_DIM), jnp.float32
        ),
        scratch_types=[
            pltpu.VMEM((2, CHUNK), jnp.int32),
            pltpu.VMEM((2, CHUNK, EMBED_DIM), jnp.float32),
            pltpu.SemaphoreType.DMA,
            pltpu.SemaphoreType.DMA,
            pltpu.SemaphoreType.DMA,
            pltpu.SemaphoreType.DMA,
        ],
    )
    def k(xt_hbm, t_hbm, out_hbm, xcol, rowbuf, xsem, gsem, gsem2, osem):
        wid = lax.axis_index("s") * NC + lax.axis_index("c")
        b0 = wid * BROWS_PER_W

        def fire(f, s, buf):
            sem = gsem if buf == 0 else gsem2

            def row_body(j, carry):
                xv = xcol[buf, pl.ds(j * LANES, LANES)]
                cv = lax.max(
                    lax.min(xv, FIELD_SIZE - 1), 0
                ) + f * FIELD_SIZE
                for l in range(LANES):
                    r = jnp.squeeze(lax.slice(cv, (l,), (l + 1,)))
                    pltpu.async_copy(
                        t_hbm.at[r], rowbuf.at[buf, j * LANES + l], sem
                    )
                return carry

            lax.fori_loop(0, CHUNK // LANES, row_body, 0)

        def drain(f, s, buf):
            sem = gsem if buf == 0 else gsem2

            def wait_body(i, carry):
                pltpu.make_async_copy(
                    t_hbm.at[0], rowbuf.at[buf, 0], sem
                ).wait()
                return carry

            lax.fori_loop(0, CHUNK, wait_body, 0)
            pltpu.async_copy(
                rowbuf.at[buf],
                out_hbm.at[pl.ds(b0 + s * CHUNK, CHUNK), f],
                osem,
            ).wait()

        def field_body(f, carry):
            pltpu.async_copy(
                xt_hbm.at[f, pl.ds(b0, CHUNK)], xcol.at[0], xsem
            ).wait()
            fire(f, 0, 0)
            pltpu.async_copy(
                xt_hbm.at[f, pl.ds(b0 + CHUNK, CHUNK)], xcol.at[1], xsem
            ).wait()
            fire(f, 1, 1)
            drain(f, 0, 0)
            drain(f, 1, 1)
            return carry

        lax.fori_loop(0, NUM_FIELDS, field_body, 0)

    return k(xt, table)


def kernel(x, table):
    out = _sc_gather(x.T, table)
    return out
